# baseline (speedup 1.0000x reference)
import dataclasses
import functools

import jax
import jax.numpy as jnp
from jax import lax
from jax.experimental import pallas as pl
from jax.experimental.pallas import tpu as pltpu
from jax.experimental.pallas import tpu_sc as plsc

NC = 2
NS = 16
NT = NC * NS
BB = 128
LCAP = 10240
SEGW = 3392
NREL = 3456
DUMPR = 3408
HSZ = NREL * 16
ACHUNK = 2000


def _mesh():
    return plsc.VectorSubcoreMesh(
        core_axis_name="c", subcore_axis_name="s", num_cores=NC, num_subcores=NS
    )


def _compiler_params():
    cp = pltpu.CompilerParams()
    if "needs_layout_passes" in pltpu.CompilerParams.__dataclass_fields__:
        cp = dataclasses.replace(cp, needs_layout_passes=False)
    return cp


def _sc_partition(e, src, dst, sgn):
    ept = e // NT
    achunks = ept // ACHUNK

    @functools.partial(
        pl.kernel,
        out_type=[
            jax.ShapeDtypeStruct((6 * NT * LCAP,), jnp.int32),
            jax.ShapeDtypeStruct((6 * NT * LCAP,), jnp.int32),
            jax.ShapeDtypeStruct((6 * NT * 16,), jnp.int32),
        ],
        mesh=_mesh(),
        scratch_types=[
            pltpu.VMEM((ACHUNK,), jnp.int32),
            pltpu.VMEM((ACHUNK,), jnp.int32),
            pltpu.VMEM((ACHUNK,), jnp.float32),
            pltpu.VMEM((LCAP + 16,), jnp.int32),
            pltpu.VMEM((LCAP + 16,), jnp.int32),
            pltpu.VMEM((LCAP + 16,), jnp.int32),
            pltpu.VMEM((LCAP + 16,), jnp.int32),
            pltpu.VMEM((LCAP + 16,), jnp.int32),
            pltpu.VMEM((LCAP + 16,), jnp.int32),
            pltpu.VMEM((LCAP + 16,), jnp.int32),
            pltpu.VMEM((LCAP + 16,), jnp.int32),
            pltpu.VMEM((LCAP + 16,), jnp.int32),
            pltpu.VMEM((LCAP + 16,), jnp.int32),
            pltpu.VMEM((LCAP + 16,), jnp.int32),
            pltpu.VMEM((LCAP + 16,), jnp.int32),
            pltpu.VMEM((16,), jnp.int32),
        ],
        compiler_params=_compiler_params(),
    )
    def part_kernel(src_hbm, dst_hbm, sgn_hbm, lsrc_out, ldst_out, cnt_out,
                    ssrc, sdst, ssgn, s0, s1, s2, s3, s4, s5,
                    d0, d1, d2, d3, d4, d5, cntv):
        c = lax.axis_index("c")
        s = lax.axis_index("s")
        g = c * NS + s

        slists = (s0, s1, s2, s3, s4, s5)
        dlists = (d0, d1, d2, d3, d4, d5)

        zsrc = jnp.zeros((16,), jnp.int32)
        zdst = jnp.full((16,), DUMPR, jnp.int32)

        @pl.loop(0, LCAP, step=16)
        def _(i):
            for q in range(6):
                slists[q][pl.ds(i, 16)] = zsrc
                dlists[q][pl.ds(i, 16)] = zdst

        iota16 = lax.iota(jnp.int32, 16)
        tidx = LCAP + iota16
        zero = jnp.int32(0)

        @pl.loop(0, achunks, init_carry=(zero,) * 6)
        def counts(t, carry):
            pltpu.sync_copy(src_hbm.at[pl.ds(g * ept + t * ACHUNK, ACHUNK)],
                            ssrc)
            pltpu.sync_copy(dst_hbm.at[pl.ds(g * ept + t * ACHUNK, ACHUNK)],
                            sdst)
            pltpu.sync_copy(sgn_hbm.at[pl.ds(g * ept + t * ACHUNK, ACHUNK)],
                            ssgn)

            @pl.loop(0, ACHUNK // 16, init_carry=carry)
            def inner(i, icarry):
                sv = ssrc[pl.ds(i * 16, 16)]
                dv = sdst[pl.ds(i * 16, 16)]
                mp = ssgn[pl.ds(i * 16, 16)] >= 0.0
                m1 = dv >= SEGW
                m2 = dv >= 2 * SEGW
                rel = dv - jnp.where(m2, 2 * SEGW, jnp.where(m1, SEGW, 0))
                third = m1.astype(jnp.int32) + m2.astype(jnp.int32)
                out = []
                for q in range(6):
                    mq = jnp.logical_and(
                        mp if q < 3 else jnp.logical_not(mp),
                        third == (q % 3))
                    mi = mq.astype(jnp.int32)
                    incl = plsc.cumsum(mi)
                    dest = jnp.where(mq, icarry[q] + incl - mi, tidx)
                    plsc.store_scatter(slists[q], [dest], sv)
                    plsc.store_scatter(dlists[q], [dest], rel)
                    out.append(icarry[q] + jnp.max(incl))
                return tuple(out)

            return inner

        for q in range(6):
            base = (q * NT + g) * LCAP
            pltpu.sync_copy(slists[q].at[pl.ds(0, LCAP)],
                            lsrc_out.at[pl.ds(base, LCAP)])
            pltpu.sync_copy(dlists[q].at[pl.ds(0, LCAP)],
                            ldst_out.at[pl.ds(base, LCAP)])
            nbq = (counts[q] + (BB - 1)) // BB
            nbq = ((nbq + 1) // 2) * 2
            cntv[...] = jnp.full((16,), 0, jnp.int32) + nbq
            pltpu.sync_copy(cntv, cnt_out.at[pl.ds((q * NT + g) * 16, 16)])

    return part_kernel(src, dst, sgn)


def _sc_accumulate(n, dfull, feat, lsrc, ldst, cnts):
    acc_slice = NREL // NS

    @functools.partial(
        pl.kernel,
        out_type=[
            jax.ShapeDtypeStruct((NC * 3 * NREL, dfull), jnp.float32),
            jax.ShapeDtypeStruct((NT * 3 * HSZ,), jnp.float32),
        ],
        mesh=_mesh(),
        scratch_types=[
            pltpu.VMEM((BB,), jnp.int32),
            pltpu.VMEM((BB,), jnp.int32),
            pltpu.VMEM((BB,), jnp.int32),
            pltpu.VMEM((BB,), jnp.int32),
            pltpu.VMEM((BB, dfull), jnp.float32),
            pltpu.VMEM((BB, dfull), jnp.float32),
            pltpu.VMEM((16,), jnp.int32),
            pltpu.VMEM((HSZ,), jnp.float32),
            pltpu.VMEM_SHARED((NREL, dfull), jnp.float32),
            pltpu.SemaphoreType.DMA,
            pltpu.SemaphoreType.DMA,
            pltpu.SemaphoreType.DMA,
            pltpu.SemaphoreType.DMA,
            pltpu.SemaphoreType.DMA,
            pltpu.SemaphoreType.DMA,
            pltpu.SemaphoreType.DMA,
            pltpu.SemaphoreType.DMA,
        ],
        compiler_params=_compiler_params(),
    )
    def acc_kernel(feat_hbm, lsrc_hbm, ldst_hbm, cnt_hbm,
                   acc_out, deg_out,
                   srcA, srcB, dstA, dstB, rowsA, rowsB, cntv, hist,
                   acc_sh, sm0, sm1, sm2, sm3, sm4, sm5, sm6, sm7):
        c = lax.axis_index("c")
        s = lax.axis_index("s")
        g = c * NS + s
        iota16 = lax.iota(jnp.int32, 16)
        zf16 = jnp.zeros((16,), jnp.float32)
        fone = jnp.ones((16,), jnp.float32)

        @pl.loop(0, 3)
        def _(h):
            @pl.loop(0, BB)
            def _(r):
                @pl.loop(0, dfull, step=16)
                def _(q):
                    rowsA[r, pl.ds(q, 16)] = zf16

            @pl.loop(0, acc_slice // BB)
            def _(i):
                pltpu.sync_copy(
                    rowsA, acc_sh.at[pl.ds(s * acc_slice + i * BB, BB)])

            rem = acc_slice % BB
            if rem:
                pltpu.sync_copy(
                    rowsA.at[pl.ds(0, rem)],
                    acc_sh.at[pl.ds(s * acc_slice + acc_slice - rem, rem)])

            @pl.loop(0, HSZ, step=16)
            def _(i):
                hist[pl.ds(i, 16)] = zf16

            plsc.subcore_barrier()

            @pl.loop(0, 2)
            def _(l):
                g2 = 2 * s + l
                lidx = (c * 3 + h) * NT + g2
                lbase = lidx * LCAP
                pltpu.sync_copy(cnt_hbm.at[pl.ds(lidx * 16, 16)], cntv)
                nb = jnp.max(cntv[...])

                @pl.loop(0, nb)
                def _(k):
                    pltpu.sync_copy(
                        lsrc_hbm.at[pl.ds(lbase + k * BB, BB)], srcA)
                    pltpu.sync_copy(
                        ldst_hbm.at[pl.ds(lbase + k * BB, BB)], dstA)
                    ga = pltpu.async_copy(feat_hbm.at[srcA], rowsA, sm0)

                    @pl.loop(0, BB // 16)
                    def _(j):
                        ra = dstA[pl.ds(j * 16, 16)]
                        plsc.addupdate_scatter(hist, [ra * 16 + iota16],
                                               fone)

                    ga.wait()
                    pltpu.sync_copy(rowsA, acc_sh.at[dstA], add=True)

            pltpu.sync_copy(hist, deg_out.at[pl.ds((g * 3 + h) * HSZ, HSZ)])
            plsc.subcore_barrier()

            pltpu.sync_copy(
                acc_sh.at[pl.ds(s * acc_slice, acc_slice)],
                acc_out.at[pl.ds((c * 3 + h) * NREL + s * acc_slice,
                                 acc_slice)])
            plsc.subcore_barrier()

    return acc_kernel(feat, lsrc, ldst, cnts)


def _tc_body(d, pos, neg, hist, x, wc, bb, o):
    deg = jnp.sum(hist[...], axis=(0, 1, 3))[:, None]
    denom = jnp.maximum(deg, 1.0)
    w = wc[...]
    dot = functools.partial(jnp.dot, precision=lax.Precision.HIGHEST,
                            preferred_element_type=jnp.float32)
    y = dot(pos[0][0], w[0:d]) + dot(neg[0][0], w[d:2 * d])
    y = y / denom + dot(x[...], w[2 * d:3 * d]) + bb[...]
    n2 = jnp.sum(y * y, axis=1, keepdims=True)
    o[...] = y * lax.rsqrt(jnp.maximum(n2, 1e-24))


def _tc_combine(n, d, dout, acc, deg, feature, wc, bb):
    bn = 848
    nb0 = SEGW // bn
    grid = ((n + bn - 1) // bn,)
    acc4 = acc.reshape(NC, 3, NREL, d)
    hist = deg.reshape(NT, 3, NREL, 16)
    return pl.pallas_call(
        functools.partial(_tc_body, d),
        grid=grid,
        in_specs=[
            pl.BlockSpec((1, 1, bn, d),
                         lambda j: (0, j // nb0, j % nb0, 0)),
            pl.BlockSpec((1, 1, bn, d),
                         lambda j: (1, j // nb0, j % nb0, 0)),
            pl.BlockSpec((NT, 1, bn, 16),
                         lambda j: (0, j // nb0, j % nb0, 0)),
            pl.BlockSpec((bn, d), lambda j: (j, 0)),
            pl.BlockSpec((3 * d, dout), lambda j: (0, 0)),
            pl.BlockSpec((1, dout), lambda j: (0, 0)),
        ],
        out_specs=pl.BlockSpec((bn, dout), lambda j: (j, 0)),
        out_shape=jax.ShapeDtypeStruct((n, dout), jnp.float32),
    )(acc4, acc4, hist, feature, wc, bb)


def kernel(feature, edge_index, edge_sign, W, b):
    n, d = feature.shape
    e = edge_index.shape[1]
    dout = W.shape[0]

    src = edge_index[0].astype(jnp.int32)
    dst = edge_index[1].astype(jnp.int32)

    lsrc, ldst, cnts = _sc_partition(e, src, dst, edge_sign)
    acc, deg = _sc_accumulate(n, d, feature, lsrc, ldst, cnts)

    wc = W.T
    bb = b.reshape(1, dout)
    return _tc_combine(n, d, dout, acc, deg, feature, wc, bb)


# speedup vs baseline: 1.3661x; 1.3661x over previous
import dataclasses
import functools

import jax
import jax.numpy as jnp
from jax import lax
from jax.experimental import pallas as pl
from jax.experimental.pallas import tpu as pltpu
from jax.experimental.pallas import tpu_sc as plsc

NC = 2
NS = 16
NT = NC * NS
BB = 128
LCAP = 10240
SEGW = 3392
NREL = 3456
DUMPR = 3408
HSZ = NREL * 16
ACHUNK = 2000


def _mesh():
    return plsc.VectorSubcoreMesh(
        core_axis_name="c", subcore_axis_name="s", num_cores=NC, num_subcores=NS
    )


def _compiler_params():
    cp = pltpu.CompilerParams()
    if "needs_layout_passes" in pltpu.CompilerParams.__dataclass_fields__:
        cp = dataclasses.replace(cp, needs_layout_passes=False)
    return cp


def _sc_partition(e, src, dst, sgn):
    ept = e // NT
    achunks = ept // ACHUNK

    @functools.partial(
        pl.kernel,
        out_type=[
            jax.ShapeDtypeStruct((6 * NT * LCAP,), jnp.int32),
            jax.ShapeDtypeStruct((6 * NT * LCAP,), jnp.int32),
            jax.ShapeDtypeStruct((6 * NT * 16,), jnp.int32),
        ],
        mesh=_mesh(),
        scratch_types=[
            pltpu.VMEM((ACHUNK,), jnp.int32),
            pltpu.VMEM((ACHUNK,), jnp.int32),
            pltpu.VMEM((ACHUNK,), jnp.float32),
            pltpu.VMEM((LCAP + 16,), jnp.int32),
            pltpu.VMEM((LCAP + 16,), jnp.int32),
            pltpu.VMEM((LCAP + 16,), jnp.int32),
            pltpu.VMEM((LCAP + 16,), jnp.int32),
            pltpu.VMEM((LCAP + 16,), jnp.int32),
            pltpu.VMEM((LCAP + 16,), jnp.int32),
            pltpu.VMEM((LCAP + 16,), jnp.int32),
            pltpu.VMEM((LCAP + 16,), jnp.int32),
            pltpu.VMEM((LCAP + 16,), jnp.int32),
            pltpu.VMEM((LCAP + 16,), jnp.int32),
            pltpu.VMEM((LCAP + 16,), jnp.int32),
            pltpu.VMEM((LCAP + 16,), jnp.int32),
            pltpu.VMEM((16,), jnp.int32),
        ],
        compiler_params=_compiler_params(),
    )
    def part_kernel(src_hbm, dst_hbm, sgn_hbm, lsrc_out, ldst_out, cnt_out,
                    ssrc, sdst, ssgn, s0, s1, s2, s3, s4, s5,
                    d0, d1, d2, d3, d4, d5, cntv):
        c = lax.axis_index("c")
        s = lax.axis_index("s")
        g = c * NS + s

        slists = (s0, s1, s2, s3, s4, s5)
        dlists = (d0, d1, d2, d3, d4, d5)

        zsrc = jnp.zeros((16,), jnp.int32)
        zdst = jnp.full((16,), DUMPR, jnp.int32)

        @pl.loop(0, LCAP, step=16)
        def _(i):
            for q in range(6):
                slists[q][pl.ds(i, 16)] = zsrc
                dlists[q][pl.ds(i, 16)] = zdst

        iota16 = lax.iota(jnp.int32, 16)
        tidx = LCAP + iota16
        zero = jnp.int32(0)

        @pl.loop(0, achunks, init_carry=(zero,) * 6)
        def counts(t, carry):
            pltpu.sync_copy(src_hbm.at[pl.ds(g * ept + t * ACHUNK, ACHUNK)],
                            ssrc)
            pltpu.sync_copy(dst_hbm.at[pl.ds(g * ept + t * ACHUNK, ACHUNK)],
                            sdst)
            pltpu.sync_copy(sgn_hbm.at[pl.ds(g * ept + t * ACHUNK, ACHUNK)],
                            ssgn)

            @pl.loop(0, ACHUNK // 16, init_carry=carry)
            def inner(i, icarry):
                sv = ssrc[pl.ds(i * 16, 16)]
                dv = sdst[pl.ds(i * 16, 16)]
                mp = ssgn[pl.ds(i * 16, 16)] >= 0.0
                m1 = dv >= SEGW
                m2 = dv >= 2 * SEGW
                rel = dv - jnp.where(m2, 2 * SEGW, jnp.where(m1, SEGW, 0))
                third = m1.astype(jnp.int32) + m2.astype(jnp.int32)
                out = []
                for q in range(6):
                    mq = jnp.logical_and(
                        mp if q < 3 else jnp.logical_not(mp),
                        third == (q % 3))
                    mi = mq.astype(jnp.int32)
                    incl = plsc.cumsum(mi)
                    dest = jnp.where(mq, icarry[q] + incl - mi, tidx)
                    plsc.store_scatter(slists[q], [dest], sv)
                    plsc.store_scatter(dlists[q], [dest], rel)
                    out.append(icarry[q] + jnp.max(incl))
                return tuple(out)

            return inner

        for q in range(6):
            base = (q * NT + g) * LCAP
            pltpu.sync_copy(slists[q].at[pl.ds(0, LCAP)],
                            lsrc_out.at[pl.ds(base, LCAP)])
            pltpu.sync_copy(dlists[q].at[pl.ds(0, LCAP)],
                            ldst_out.at[pl.ds(base, LCAP)])
            nbq = (counts[q] + (BB - 1)) // BB
            cntv[...] = jnp.full((16,), 0, jnp.int32) + nbq
            pltpu.sync_copy(cntv, cnt_out.at[pl.ds((q * NT + g) * 16, 16)])

    return part_kernel(src, dst, sgn)


def _sc_accumulate(n, dfull, feat, lsrc, ldst, cnts):
    acc_slice = NREL // NS

    @functools.partial(
        pl.kernel,
        out_type=[
            jax.ShapeDtypeStruct((NC * 3 * NREL, dfull), jnp.float32),
            jax.ShapeDtypeStruct((NT * 3 * HSZ,), jnp.float32),
        ],
        mesh=_mesh(),
        scratch_types=[
            pltpu.VMEM((BB,), jnp.int32),
            pltpu.VMEM((BB,), jnp.int32),
            pltpu.VMEM((BB,), jnp.int32),
            pltpu.VMEM((BB,), jnp.int32),
            pltpu.VMEM((BB, dfull), jnp.float32),
            pltpu.VMEM((BB, dfull), jnp.float32),
            pltpu.VMEM((16,), jnp.int32),
            pltpu.VMEM((HSZ,), jnp.float32),
            pltpu.VMEM_SHARED((NREL, dfull), jnp.float32),
            pltpu.SemaphoreType.DMA,
            pltpu.SemaphoreType.DMA,
            pltpu.SemaphoreType.DMA,
            pltpu.SemaphoreType.DMA,
            pltpu.SemaphoreType.DMA,
            pltpu.SemaphoreType.DMA,
            pltpu.SemaphoreType.DMA,
            pltpu.SemaphoreType.DMA,
        ],
        compiler_params=_compiler_params(),
    )
    def acc_kernel(feat_hbm, lsrc_hbm, ldst_hbm, cnt_hbm,
                   acc_out, deg_out,
                   srcA, srcB, dstA, dstB, rowsA, rowsB, cntv, hist,
                   acc_sh, sm0, sm1, sm2, sm3, sm4, sm5, sm6, sm7):
        c = lax.axis_index("c")
        s = lax.axis_index("s")
        g = c * NS + s
        iota16 = lax.iota(jnp.int32, 16)
        zf16 = jnp.zeros((16,), jnp.float32)
        fone = jnp.ones((16,), jnp.float32)

        @pl.loop(0, 3)
        def _(h):
            @pl.loop(0, BB)
            def _(r):
                @pl.loop(0, dfull, step=16)
                def _(q):
                    rowsA[r, pl.ds(q, 16)] = zf16

            @pl.loop(0, acc_slice // BB)
            def _(i):
                pltpu.sync_copy(
                    rowsA, acc_sh.at[pl.ds(s * acc_slice + i * BB, BB)])

            rem = acc_slice % BB
            if rem:
                pltpu.sync_copy(
                    rowsA.at[pl.ds(0, rem)],
                    acc_sh.at[pl.ds(s * acc_slice + acc_slice - rem, rem)])

            @pl.loop(0, HSZ, step=16)
            def _(i):
                hist[pl.ds(i, 16)] = zf16

            plsc.subcore_barrier()

            @pl.loop(0, 2)
            def _(l):
                g2 = 2 * s + l
                lidx = (c * 3 + h) * NT + g2
                lbase = lidx * LCAP
                pltpu.sync_copy(cnt_hbm.at[pl.ds(lidx * 16, 16)], cntv)
                nb = jnp.max(cntv[...])

                @pl.loop(0, nb)
                def _(k):
                    pltpu.sync_copy(
                        lsrc_hbm.at[pl.ds(lbase + k * BB, BB)], srcA)
                    pltpu.sync_copy(
                        ldst_hbm.at[pl.ds(lbase + k * BB, BB)], dstA)
                    pltpu.async_copy(feat_hbm.at[srcA], rowsA, sm0).wait()
                    pltpu.sync_copy(rowsA, acc_sh.at[dstA], add=True)

                    @pl.loop(0, BB // 16)
                    def _(j):
                        ra = dstA[pl.ds(j * 16, 16)]
                        plsc.addupdate_scatter(hist, [ra * 16 + iota16],
                                               fone)

            pltpu.sync_copy(hist, deg_out.at[pl.ds((g * 3 + h) * HSZ, HSZ)])
            plsc.subcore_barrier()

            pltpu.sync_copy(
                acc_sh.at[pl.ds(s * acc_slice, acc_slice)],
                acc_out.at[pl.ds((c * 3 + h) * NREL + s * acc_slice,
                                 acc_slice)])
            plsc.subcore_barrier()

    return acc_kernel(feat, lsrc, ldst, cnts)


def _tc_body(d, pos, neg, hist, x, wc, bb, o):
    deg = jnp.sum(hist[...], axis=(0, 1, 3))[:, None]
    denom = jnp.maximum(deg, 1.0)
    w = wc[...]
    dot = functools.partial(jnp.dot, precision=lax.Precision.HIGHEST,
                            preferred_element_type=jnp.float32)
    y = dot(pos[0][0], w[0:d]) + dot(neg[0][0], w[d:2 * d])
    y = y / denom + dot(x[...], w[2 * d:3 * d]) + bb[...]
    n2 = jnp.sum(y * y, axis=1, keepdims=True)
    o[...] = y * lax.rsqrt(jnp.maximum(n2, 1e-24))


def _tc_combine(n, d, dout, acc, deg, feature, wc, bb):
    bn = 848
    nb0 = SEGW // bn
    grid = ((n + bn - 1) // bn,)
    acc4 = acc.reshape(NC, 3, NREL, d)
    hist = deg.reshape(NT, 3, NREL, 16)
    return pl.pallas_call(
        functools.partial(_tc_body, d),
        grid=grid,
        in_specs=[
            pl.BlockSpec((1, 1, bn, d),
                         lambda j: (0, j // nb0, j % nb0, 0)),
            pl.BlockSpec((1, 1, bn, d),
                         lambda j: (1, j // nb0, j % nb0, 0)),
            pl.BlockSpec((NT, 1, bn, 16),
                         lambda j: (0, j // nb0, j % nb0, 0)),
            pl.BlockSpec((bn, d), lambda j: (j, 0)),
            pl.BlockSpec((3 * d, dout), lambda j: (0, 0)),
            pl.BlockSpec((1, dout), lambda j: (0, 0)),
        ],
        out_specs=pl.BlockSpec((bn, dout), lambda j: (j, 0)),
        out_shape=jax.ShapeDtypeStruct((n, dout), jnp.float32),
    )(acc4, acc4, hist, feature, wc, bb)


def kernel(feature, edge_index, edge_sign, W, b):
    n, d = feature.shape
    e = edge_index.shape[1]
    dout = W.shape[0]

    src = edge_index[0].astype(jnp.int32)
    dst = edge_index[1].astype(jnp.int32)

    lsrc, ldst, cnts = _sc_partition(e, src, dst, edge_sign)
    acc, deg = _sc_accumulate(n, d, feature, lsrc, ldst, cnts)

    wc = W.T
    bb = b.reshape(1, dout)
    return _tc_combine(n, d, dout, acc, deg, feature, wc, bb)


# speedup vs baseline: 1.4067x; 1.0297x over previous
import dataclasses
import functools

import jax
import jax.numpy as jnp
from jax import lax
from jax.experimental import pallas as pl
from jax.experimental.pallas import tpu as pltpu
from jax.experimental.pallas import tpu_sc as plsc

NC = 2
NS = 16
NT = NC * NS
BB = 128
LCAP = 10240
SEGW = 3392
NREL = 3456
DUMPR = 3408
HSZ = NREL * 16
ACHUNK = 2000


def _mesh():
    return plsc.VectorSubcoreMesh(
        core_axis_name="c", subcore_axis_name="s", num_cores=NC, num_subcores=NS
    )


def _compiler_params():
    cp = pltpu.CompilerParams()
    if "needs_layout_passes" in pltpu.CompilerParams.__dataclass_fields__:
        cp = dataclasses.replace(cp, needs_layout_passes=False)
    return cp


def _sc_partition(e, src, dst, sgn):
    ept = e // NT
    achunks = ept // ACHUNK

    @functools.partial(
        pl.kernel,
        out_type=[
            jax.ShapeDtypeStruct((6 * NT * LCAP,), jnp.int32),
            jax.ShapeDtypeStruct((6 * NT * LCAP,), jnp.int32),
            jax.ShapeDtypeStruct((6 * NT * 16,), jnp.int32),
        ],
        mesh=_mesh(),
        scratch_types=[
            pltpu.VMEM((ACHUNK,), jnp.int32),
            pltpu.VMEM((ACHUNK,), jnp.int32),
            pltpu.VMEM((ACHUNK,), jnp.float32),
            pltpu.VMEM((LCAP + 16,), jnp.int32),
            pltpu.VMEM((LCAP + 16,), jnp.int32),
            pltpu.VMEM((LCAP + 16,), jnp.int32),
            pltpu.VMEM((LCAP + 16,), jnp.int32),
            pltpu.VMEM((LCAP + 16,), jnp.int32),
            pltpu.VMEM((LCAP + 16,), jnp.int32),
            pltpu.VMEM((LCAP + 16,), jnp.int32),
            pltpu.VMEM((LCAP + 16,), jnp.int32),
            pltpu.VMEM((LCAP + 16,), jnp.int32),
            pltpu.VMEM((LCAP + 16,), jnp.int32),
            pltpu.VMEM((LCAP + 16,), jnp.int32),
            pltpu.VMEM((LCAP + 16,), jnp.int32),
            pltpu.VMEM((16,), jnp.int32),
        ],
        compiler_params=_compiler_params(),
    )
    def part_kernel(src_hbm, dst_hbm, sgn_hbm, lsrc_out, ldst_out, cnt_out,
                    ssrc, sdst, ssgn, s0, s1, s2, s3, s4, s5,
                    d0, d1, d2, d3, d4, d5, cntv):
        c = lax.axis_index("c")
        s = lax.axis_index("s")
        g = c * NS + s

        slists = (s0, s1, s2, s3, s4, s5)
        dlists = (d0, d1, d2, d3, d4, d5)

        zsrc = jnp.zeros((16,), jnp.int32)
        zdst = jnp.full((16,), DUMPR, jnp.int32)

        @pl.loop(0, LCAP, step=16)
        def _(i):
            for q in range(6):
                slists[q][pl.ds(i, 16)] = zsrc
                dlists[q][pl.ds(i, 16)] = zdst

        iota16 = lax.iota(jnp.int32, 16)
        tidx = LCAP + iota16
        zero = jnp.int32(0)

        @pl.loop(0, achunks, init_carry=(zero,) * 6)
        def counts(t, carry):
            pltpu.sync_copy(src_hbm.at[pl.ds(g * ept + t * ACHUNK, ACHUNK)],
                            ssrc)
            pltpu.sync_copy(dst_hbm.at[pl.ds(g * ept + t * ACHUNK, ACHUNK)],
                            sdst)
            pltpu.sync_copy(sgn_hbm.at[pl.ds(g * ept + t * ACHUNK, ACHUNK)],
                            ssgn)

            @pl.loop(0, ACHUNK // 16, init_carry=carry)
            def inner(i, icarry):
                sv = ssrc[pl.ds(i * 16, 16)]
                dv = sdst[pl.ds(i * 16, 16)]
                mp = ssgn[pl.ds(i * 16, 16)] >= 0.0
                m1 = dv >= SEGW
                m2 = dv >= 2 * SEGW
                rel = dv - jnp.where(m2, 2 * SEGW, jnp.where(m1, SEGW, 0))
                third = m1.astype(jnp.int32) + m2.astype(jnp.int32)
                out = []
                for q in range(6):
                    mq = jnp.logical_and(
                        mp if q < 3 else jnp.logical_not(mp),
                        third == (q % 3))
                    mi = mq.astype(jnp.int32)
                    incl = plsc.cumsum(mi)
                    dest = jnp.where(mq, icarry[q] + incl - mi, tidx)
                    plsc.store_scatter(slists[q], [dest], sv)
                    plsc.store_scatter(dlists[q], [dest], rel)
                    out.append(icarry[q] + jnp.max(incl))
                return tuple(out)

            return inner

        for q in range(6):
            base = (q * NT + g) * LCAP
            pltpu.sync_copy(slists[q].at[pl.ds(0, LCAP)],
                            lsrc_out.at[pl.ds(base, LCAP)])
            pltpu.sync_copy(dlists[q].at[pl.ds(0, LCAP)],
                            ldst_out.at[pl.ds(base, LCAP)])
            nbq = (counts[q] + (BB - 1)) // BB
            cntv[...] = jnp.full((16,), 0, jnp.int32) + nbq
            pltpu.sync_copy(cntv, cnt_out.at[pl.ds((q * NT + g) * 16, 16)])

    return part_kernel(src, dst, sgn)


def _sc_accumulate(n, dfull, feat, lsrc, ldst, cnts, h):
    acc_slice = NREL // NS

    @functools.partial(
        pl.kernel,
        out_type=[
            jax.ShapeDtypeStruct((NC * NREL, dfull), jnp.float32),
            jax.ShapeDtypeStruct((NT * HSZ,), jnp.float32),
        ],
        mesh=_mesh(),
        scratch_types=[
            pltpu.VMEM((BB,), jnp.int32),
            pltpu.VMEM((BB,), jnp.int32),
            pltpu.VMEM((BB,), jnp.int32),
            pltpu.VMEM((BB,), jnp.int32),
            pltpu.VMEM((BB, dfull), jnp.float32),
            pltpu.VMEM((BB, dfull), jnp.float32),
            pltpu.VMEM((16,), jnp.int32),
            pltpu.VMEM((HSZ,), jnp.float32),
            pltpu.VMEM_SHARED((NREL, dfull), jnp.float32),
            pltpu.SemaphoreType.DMA,
            pltpu.SemaphoreType.DMA,
            pltpu.SemaphoreType.DMA,
            pltpu.SemaphoreType.DMA,
            pltpu.SemaphoreType.DMA,
            pltpu.SemaphoreType.DMA,
            pltpu.SemaphoreType.DMA,
            pltpu.SemaphoreType.DMA,
        ],
        compiler_params=_compiler_params(),
        name=f"sc_accumulate_h{h}",
    )
    def acc_kernel(feat_hbm, lsrc_hbm, ldst_hbm, cnt_hbm,
                   acc_out, deg_out,
                   srcA, srcB, dstA, dstB, rowsA, rowsB, cntv, hist,
                   acc_sh, sm0, sm1, sm2, sm3, sm4, sm5, sm6, sm7):
        c = lax.axis_index("c")
        s = lax.axis_index("s")
        g = c * NS + s
        iota16 = lax.iota(jnp.int32, 16)
        zf16 = jnp.zeros((16,), jnp.float32)
        fone = jnp.ones((16,), jnp.float32)

        if True:
            @pl.loop(0, BB)
            def _(r):
                @pl.loop(0, dfull, step=16)
                def _(q):
                    rowsA[r, pl.ds(q, 16)] = zf16

            @pl.loop(0, acc_slice // BB)
            def _(i):
                pltpu.sync_copy(
                    rowsA, acc_sh.at[pl.ds(s * acc_slice + i * BB, BB)])

            rem = acc_slice % BB
            if rem:
                pltpu.sync_copy(
                    rowsA.at[pl.ds(0, rem)],
                    acc_sh.at[pl.ds(s * acc_slice + acc_slice - rem, rem)])

            @pl.loop(0, HSZ, step=16)
            def _(i):
                hist[pl.ds(i, 16)] = zf16

            plsc.subcore_barrier()

            @pl.loop(0, 2)
            def _(l):
                g2 = 2 * s + l
                lidx = (c * 3 + h) * NT + g2
                lbase = lidx * LCAP
                pltpu.sync_copy(cnt_hbm.at[pl.ds(lidx * 16, 16)], cntv)
                nb = jnp.max(cntv[...])

                @pl.loop(0, nb)
                def _(k):
                    pltpu.sync_copy(
                        lsrc_hbm.at[pl.ds(lbase + k * BB, BB)], srcA)
                    pltpu.sync_copy(
                        ldst_hbm.at[pl.ds(lbase + k * BB, BB)], dstA)
                    pltpu.async_copy(feat_hbm.at[srcA], rowsA, sm0).wait()
                    pltpu.sync_copy(rowsA, acc_sh.at[dstA], add=True)

                    @pl.loop(0, BB // 16)
                    def _(j):
                        ra = dstA[pl.ds(j * 16, 16)]
                        plsc.addupdate_scatter(hist, [ra * 16 + iota16],
                                               fone)

            pltpu.sync_copy(hist, deg_out.at[pl.ds(g * HSZ, HSZ)])
            plsc.subcore_barrier()

            pltpu.sync_copy(
                acc_sh.at[pl.ds(s * acc_slice, acc_slice)],
                acc_out.at[pl.ds(c * NREL + s * acc_slice, acc_slice)])

    return acc_kernel(feat, lsrc, ldst, cnts)


def _tc_body(d, pos, neg, hist, x, wc, bb, o):
    deg = jnp.sum(hist[...], axis=(0, 2))[:, None]
    denom = jnp.maximum(deg, 1.0)
    w = wc[...]
    dot = functools.partial(jnp.dot, precision=lax.Precision.HIGHEST,
                            preferred_element_type=jnp.float32)
    y = dot(pos[0], w[0:d]) + dot(neg[0], w[d:2 * d])
    y = y / denom + dot(x[...], w[2 * d:3 * d]) + bb[...]
    n2 = jnp.sum(y * y, axis=1, keepdims=True)
    o[...] = y * lax.rsqrt(jnp.maximum(n2, 1e-24))


def _tc_combine(n, d, dout, acc, deg, feature, wc, bb, h):
    bn = 848
    rows = min(SEGW, n - h * SEGW)
    grid = ((rows + bn - 1) // bn,)
    acc4 = acc.reshape(NC, NREL, d)
    hist = deg.reshape(NT, NREL, 16)
    off = h * (SEGW // bn)
    return pl.pallas_call(
        functools.partial(_tc_body, d),
        grid=grid,
        in_specs=[
            pl.BlockSpec((1, bn, d), lambda j: (0, j, 0)),
            pl.BlockSpec((1, bn, d), lambda j: (1, j, 0)),
            pl.BlockSpec((NT, bn, 16), lambda j: (0, j, 0)),
            pl.BlockSpec((bn, d), lambda j: (j + off, 0)),
            pl.BlockSpec((3 * d, dout), lambda j: (0, 0)),
            pl.BlockSpec((1, dout), lambda j: (0, 0)),
        ],
        out_specs=pl.BlockSpec((bn, dout), lambda j: (j, 0)),
        out_shape=jax.ShapeDtypeStruct((rows, dout), jnp.float32),
    )(acc4, acc4, hist, feature, wc, bb)


def kernel(feature, edge_index, edge_sign, W, b):
    n, d = feature.shape
    e = edge_index.shape[1]
    dout = W.shape[0]

    src = edge_index[0].astype(jnp.int32)
    dst = edge_index[1].astype(jnp.int32)

    lsrc, ldst, cnts = _sc_partition(e, src, dst, edge_sign)
    wc = W.T
    bb = b.reshape(1, dout)
    outs = []
    for h in range(3):
        acc, deg = _sc_accumulate(n, d, feature, lsrc, ldst, cnts, h)
        outs.append(_tc_combine(n, d, dout, acc, deg, feature, wc, bb, h))
    return jnp.concatenate(outs, axis=0)
